# baseline (speedup 1.0000x reference)
import jax
import jax.numpy as jnp
from jax.experimental import pallas as pl
from jax.experimental.pallas import tpu as pltpu

NBUF = 3
BLOCK_ROWS = 400


def _copy_block(adj_hbm, abuf, sem, blk, slot, bm):
    return pltpu.make_async_copy(
        adj_hbm.at[pl.ds(blk * bm, bm), :],
        abuf.at[slot],
        sem.at[slot],
    )


def _make_kernel(bm, nbuf):
    def _kern(x_ref, wt_ref, b_ref, adj_hbm, o_ref, abuf, sem):
        i = pl.program_id(0)
        nblk = pl.num_programs(0)

        @pl.when(i == 0)
        def _prologue():
            for j in range(nbuf - 1):
                _copy_block(adj_hbm, abuf, sem, j, j, bm).start()

        nxt = i + nbuf - 1

        @pl.when(nxt < nblk)
        def _issue():
            slot = jax.lax.rem(nxt, nbuf)
            _copy_block(adj_hbm, abuf, sem, nxt, slot, bm).start()

        slot = jax.lax.rem(i, nbuf)
        _copy_block(adj_hbm, abuf, sem, i, slot, bm).wait()
        h = jnp.dot(abuf[slot], x_ref[...],
                    preferred_element_type=jnp.float32)
        y = jnp.dot(h, wt_ref[...],
                    preferred_element_type=jnp.float32) + b_ref[...]
        o_ref[...] = jnp.maximum(y, 0.0)

    return _kern


@jax.jit
def _run(x, adj, wt, b2):
    n, d_in = x.shape
    d_out = wt.shape[1]
    bm = BLOCK_ROWS
    assert n % bm == 0
    grid = (n // bm,)
    return pl.pallas_call(
        _make_kernel(bm, NBUF),
        grid=grid,
        in_specs=[
            pl.BlockSpec((n, d_in), lambda i: (0, 0)),
            pl.BlockSpec((d_in, d_out), lambda i: (0, 0)),
            pl.BlockSpec((1, d_out), lambda i: (0, 0)),
            pl.BlockSpec(memory_space=pl.ANY),
        ],
        out_specs=pl.BlockSpec((bm, d_out), lambda i: (i, 0)),
        out_shape=jax.ShapeDtypeStruct((n, d_out), jnp.float32),
        scratch_shapes=[
            pltpu.VMEM((NBUF, bm, n), jnp.float32),
            pltpu.SemaphoreType.DMA((NBUF,)),
        ],
        compiler_params=pltpu.CompilerParams(
            dimension_semantics=("arbitrary",),
        ),
    )(x, wt, b2, adj)


def kernel(input, adj, W, b):
    wt = W.T
    b2 = b.reshape(1, -1)
    return _run(input, adj, wt, b2)


# speedup vs baseline: 1.0495x; 1.0495x over previous
import jax
import jax.numpy as jnp
from jax.experimental import pallas as pl
from jax.experimental.pallas import tpu as pltpu

BLOCK_ROWS = 400


def _fused_gcn_kernel(x_ref, w_ref, b_ref, adj_ref, o_ref):
    h = jnp.dot(adj_ref[...], x_ref[...], preferred_element_type=jnp.float32)
    y = jax.lax.dot_general(
        h, w_ref[...],
        dimension_numbers=(((1,), (1,)), ((), ())),
        preferred_element_type=jnp.float32,
    ) + b_ref[...]
    o_ref[...] = jnp.maximum(y, 0.0)


@jax.jit
def _run(x, adj, w, b):
    n, d_in = x.shape
    d_out = w.shape[0]
    bm = BLOCK_ROWS
    assert n % bm == 0
    grid = (n // bm,)
    return pl.pallas_call(
        _fused_gcn_kernel,
        grid=grid,
        in_specs=[
            pl.BlockSpec((n, d_in), lambda i: (0, 0)),
            pl.BlockSpec((d_out, d_in), lambda i: (0, 0)),
            pl.BlockSpec((d_out,), lambda i: (0,)),
            pl.BlockSpec((bm, n), lambda i: (i, 0)),
        ],
        out_specs=pl.BlockSpec((bm, d_out), lambda i: (i, 0)),
        out_shape=jax.ShapeDtypeStruct((n, d_out), jnp.float32),
        compiler_params=pltpu.CompilerParams(
            dimension_semantics=("arbitrary",),
        ),
    )(x, w, b, adj)


def kernel(input, adj, W, b):
    return _run(input, adj, W, b)
